# baseline (speedup 1.0000x reference)
import jax
import jax.numpy as jnp
from jax import lax
from jax.experimental import pallas as pl
from jax.experimental.pallas import tpu as pltpu
from jax.experimental.pallas import tpu_sc as plsc

N_NODES_C = 10000
N_EDGES_C = 320000
D_IN = 128
NC = 2
NS = 16
NHALF = N_NODES_C // NC
ACC_ROWS = 5120
EDGES_PER_TILE = N_EDGES_C // NS
BLK = 80
NBLK = EDGES_PER_TILE // BLK
NPAIR = NBLK // 2 + 1
MCAP = (2 * NPAIR + 2) * BLK


_GDN = lax.GatherDimensionNumbers(
    offset_dims=(), collapsed_slice_dims=(0,), start_index_map=(0,))


def _splat_lane(vec, lane):
    idx = jnp.full((16, 1), lane, dtype=jnp.int32)
    return lax.gather(vec, idx, _GDN, slice_sizes=(1,),
                      mode=lax.GatherScatterMode.PROMISE_IN_BOUNDS)


CB = 25
NCHUNK = EDGES_PER_TILE // (CB * BLK)
MBUF = (CB + 2) * BLK


def _sc_body(x_hbm, src_hbm, dst_hbm, u_hbm, zeros_hbm,
             accx_out, accux_out,
             srcb_v, dstb_v, ub_v,
             dstl0_v, dstl1_v, rows0_v, rows1_v, ux_v,
             gsem0, gsem1, ssem0, ssem1, usem,
             accx_sh, accux_sh):
    c = lax.axis_index("c")
    s = lax.axis_index("s")
    slice_rows = ACC_ROWS // NS
    r0 = s * slice_rows
    pltpu.sync_copy(zeros_hbm.at[pl.ds(r0, slice_rows)],
                    accx_sh.at[pl.ds(r0, slice_rows)])
    pltpu.sync_copy(zeros_hbm.at[pl.ds(r0, slice_rows)],
                    accux_sh.at[pl.ds(r0, slice_rows)])
    base_e = s * EDGES_PER_TILE
    half_lo = c * NHALF
    for p in range((MBUF - CB * BLK) // 16):
        o = CB * BLK + p * 16
        srcb_v[pl.ds(o, 16)] = jnp.zeros((16,), jnp.int32)
        dstb_v[pl.ds(o, 16)] = jnp.zeros((16,), jnp.int32) + half_lo + NHALF
        ub_v[pl.ds(o, 16)] = jnp.zeros((16,), jnp.float32)
    plsc.subcore_barrier()

    joff = pl.multiple_of((CB + 1) * BLK, BLK)
    pltpu.async_copy(x_hbm.at[srcb_v.at[pl.ds(joff, BLK)]], rows0_v, gsem0)
    bufs = ((rows0_v, gsem0), (rows1_v, gsem1))

    def chunk(ch, carry):
        pltpu.make_async_copy(
            x_hbm.at[srcb_v.at[pl.ds(joff, BLK)]], rows0_v, gsem0).wait()
        e0 = pl.multiple_of(base_e + ch * CB * BLK, BLK)
        pltpu.sync_copy(src_hbm.at[pl.ds(e0, CB * BLK)],
                        srcb_v.at[pl.ds(0, CB * BLK)])
        pltpu.sync_copy(dst_hbm.at[pl.ds(e0, CB * BLK)],
                        dstb_v.at[pl.ds(0, CB * BLK)])
        pltpu.sync_copy(u_hbm.at[pl.ds(e0, CB * BLK)],
                        ub_v.at[pl.ds(0, CB * BLK)])
        pltpu.async_copy(x_hbm.at[srcb_v.at[pl.ds(0, BLK)]], rows0_v,
                         gsem0)

        def pair(ip, carry2):
            for b in (0, 1):
                o = 1 - b
                rows_v, gsem = bufs[b]
                rows_o, gsem_o = bufs[o]
                g = 2 * ip + b
                off = g * BLK
                pltpu.make_async_copy(
                    x_hbm.at[srcb_v.at[pl.ds(off, BLK)]], rows_v,
                    gsem).wait()
                for gg in range(BLK // 16):
                    d = dstb_v[pl.ds(off + gg * 16, 16)]
                    dl = d - half_lo
                    oob = (dl < 0) | (dl >= NHALF)
                    dstl0_v[pl.ds(gg * 16, 16)] = jnp.where(oob, NHALF, dl)
                off1 = (g + 1) * BLK
                pltpu.async_copy(x_hbm.at[srcb_v.at[pl.ds(off1, BLK)]],
                                 rows_o, gsem_o)
                for gg in range(BLK // 16):
                    uv = ub_v[pl.ds(off + gg * 16, 16)]
                    uv = jnp.clip(uv, 0.0, 1.0)
                    for l in range(16):
                        j = gg * 16 + l
                        uj = _splat_lane(uv, l)
                        for k in range(D_IN // 16):
                            ux_v[j, pl.ds(k * 16, 16)] = (
                                rows_v[j, pl.ds(k * 16, 16)] * uj)
                d1 = pltpu.async_copy(rows_v, accx_sh.at[dstl0_v], ssem0,
                                      add=True)
                d2 = pltpu.async_copy(ux_v, accux_sh.at[dstl0_v], ssem1,
                                      add=True)
                d1.wait()
                d2.wait()
            return carry2

        lax.fori_loop(0, (CB + 1) // 2, pair, 0)
        return carry

    lax.fori_loop(0, NCHUNK, chunk, 0)
    pltpu.make_async_copy(
        x_hbm.at[srcb_v.at[pl.ds(joff, BLK)]], rows0_v, gsem0).wait()
    plsc.subcore_barrier()
    pltpu.sync_copy(accx_sh.at[pl.ds(r0, slice_rows)],
                    accx_out.at[c, pl.ds(r0, slice_rows)])
    pltpu.sync_copy(accux_sh.at[pl.ds(r0, slice_rows)],
                    accux_out.at[c, pl.ds(r0, slice_rows)])


def _deg_body(dst_hbm, zeros_hbm, ones_hbm,
              deg_out,
              dstb_v, dstl0_v, dstl1_v, ones_v, sem0, sem1, deg_sh):
    c = lax.axis_index("c")
    s = lax.axis_index("s")
    slice_rows = ACC_ROWS // NS
    r0 = s * slice_rows
    pltpu.sync_copy(ones_hbm, ones_v)
    pltpu.sync_copy(zeros_hbm.at[pl.ds(r0, slice_rows)],
                    deg_sh.at[pl.ds(r0, slice_rows)])
    base_e = s * EDGES_PER_TILE
    pltpu.sync_copy(dst_hbm.at[pl.ds(base_e, EDGES_PER_TILE)],
                    dstb_v.at[pl.ds(0, EDGES_PER_TILE)])
    half_lo = c * NHALF
    for p in range((MCAP - EDGES_PER_TILE) // 16):
        o = EDGES_PER_TILE + p * 16
        dstb_v[pl.ds(o, 16)] = jnp.zeros((16,), jnp.int32) + half_lo + NHALF
    plsc.subcore_barrier()

    for gg in range(BLK // 16):
        dstl0_v[pl.ds(gg * 16, 16)] = jnp.full((16,), NHALF, jnp.int32)
        dstl1_v[pl.ds(gg * 16, 16)] = jnp.full((16,), NHALF, jnp.int32)
    pltpu.async_copy(ones_v, deg_sh.at[dstl0_v], sem0, add=True)
    pltpu.async_copy(ones_v, deg_sh.at[dstl1_v], sem1, add=True)

    bufs = ((dstl0_v, sem0), (dstl1_v, sem1))

    def pair(i2, carry):
        for b in (0, 1):
            dstl_v, sem = bufs[b]
            g = 2 * i2 + b
            off = pl.multiple_of(g * BLK, BLK)
            pltpu.make_async_copy(ones_v, deg_sh.at[dstl_v], sem).wait()
            for gg in range(BLK // 16):
                d = dstb_v[pl.ds(off + gg * 16, 16)]
                dl = d - half_lo
                oob = (dl < 0) | (dl >= NHALF)
                dstl_v[pl.ds(gg * 16, 16)] = jnp.where(oob, NHALF, dl)
            pltpu.async_copy(ones_v, deg_sh.at[dstl_v], sem, add=True)
        return carry

    lax.fori_loop(0, NPAIR, pair, 0)
    pltpu.make_async_copy(ones_v, deg_sh.at[dstl0_v], sem0).wait()
    pltpu.make_async_copy(ones_v, deg_sh.at[dstl1_v], sem1).wait()
    plsc.subcore_barrier()
    pltpu.sync_copy(deg_sh.at[pl.ds(r0, slice_rows)],
                    deg_out.at[c, pl.ds(r0, slice_rows)])


@jax.jit
def _sc_aggregate(x, src, dst, u, zeros, ones):
    mesh = plsc.VectorSubcoreMesh(core_axis_name="c", subcore_axis_name="s")
    acc_t = jax.ShapeDtypeStruct((NC, ACC_ROWS, D_IN), jnp.float32)
    main = pl.kernel(
        _sc_body,
        out_type=[acc_t, acc_t],
        mesh=mesh,
        scratch_types=[
            pltpu.VMEM((MBUF,), jnp.int32),
            pltpu.VMEM((MBUF,), jnp.int32),
            pltpu.VMEM((MBUF,), jnp.float32),
            pltpu.VMEM((BLK,), jnp.int32),
            pltpu.VMEM((BLK,), jnp.int32),
            pltpu.VMEM((BLK, D_IN), jnp.float32),
            pltpu.VMEM((BLK, D_IN), jnp.float32),
            pltpu.VMEM((BLK, D_IN), jnp.float32),
            pltpu.SemaphoreType.DMA,
            pltpu.SemaphoreType.DMA,
            pltpu.SemaphoreType.DMA,
            pltpu.SemaphoreType.DMA,
            pltpu.SemaphoreType.DMA,
            pltpu.VMEM_SHARED((ACC_ROWS, D_IN), jnp.float32),
            pltpu.VMEM_SHARED((ACC_ROWS, D_IN), jnp.float32),
        ],
    )
    accx, accux = main(x, src, dst, u, zeros)
    degk = pl.kernel(
        _deg_body,
        out_type=[acc_t],
        mesh=mesh,
        scratch_types=[
            pltpu.VMEM((MCAP,), jnp.int32),
            pltpu.VMEM((BLK,), jnp.int32),
            pltpu.VMEM((BLK,), jnp.int32),
            pltpu.VMEM((BLK, D_IN), jnp.float32),
            pltpu.SemaphoreType.DMA,
            pltpu.SemaphoreType.DMA,
            pltpu.VMEM_SHARED((ACC_ROWS, D_IN), jnp.float32),
        ],
    )
    (deg,) = degk(dst, zeros, ones)
    return accx, accux, deg


def _tc_body(sx_ref, sux_ref, deg_ref, x_ref, w0_ref, w1m0_ref, rw_ref,
             o_ref):
    sx = sx_ref[...]
    su = sux_ref[...]
    deg = deg_ref[:, 0:1]
    m = jnp.dot(sx, w0_ref[...], preferred_element_type=jnp.float32,
                precision=lax.Precision.HIGHEST)
    m += jnp.dot(su, w1m0_ref[...], preferred_element_type=jnp.float32,
                 precision=lax.Precision.HIGHEST)
    root = jnp.dot(x_ref[...], rw_ref[...], preferred_element_type=jnp.float32,
                   precision=lax.Precision.HIGHEST)
    o_ref[...] = m / jnp.maximum(deg, 1.0) + root


@jax.jit
def _tc_combine(sx, sux, deg, x, w0, w1m0, root_w):
    rows = 1000
    grid = N_NODES_C // rows
    return pl.pallas_call(
        _tc_body,
        grid=(grid,),
        in_specs=[
            pl.BlockSpec((rows, D_IN), lambda i: (i, 0)),
            pl.BlockSpec((rows, D_IN), lambda i: (i, 0)),
            pl.BlockSpec((rows, D_IN), lambda i: (i, 0)),
            pl.BlockSpec((rows, D_IN), lambda i: (i, 0)),
            pl.BlockSpec((D_IN, D_IN), lambda i: (0, 0)),
            pl.BlockSpec((D_IN, D_IN), lambda i: (0, 0)),
            pl.BlockSpec((D_IN, D_IN), lambda i: (0, 0)),
        ],
        out_specs=pl.BlockSpec((rows, D_IN), lambda i: (i, 0)),
        out_shape=jax.ShapeDtypeStruct((N_NODES_C, D_IN), jnp.float32),
    )(sx, sux, deg, x, w0, w1m0, root_w)


def kernel(node_feature, edge_index, edge_feature, W, root_W):
    src = edge_index[0].astype(jnp.int32)
    dst = edge_index[1].astype(jnp.int32)
    u = edge_feature[:, 0]
    zeros = jnp.zeros((ACC_ROWS, D_IN), jnp.float32)
    ones = jnp.ones((BLK, D_IN), jnp.float32)
    accx, accux, accdeg = _sc_aggregate(node_feature, src, dst, u, zeros,
                                        ones)
    sx = accx[:, :NHALF, :].reshape(N_NODES_C, D_IN)
    sux = accux[:, :NHALF, :].reshape(N_NODES_C, D_IN)
    deg = accdeg[:, :NHALF, :].reshape(N_NODES_C, D_IN)
    return _tc_combine(sx, sux, deg, node_feature,
                       W[0], W[1] - W[0], root_W)


# speedup vs baseline: 1.5663x; 1.5663x over previous
import jax
import jax.numpy as jnp
from jax import lax
from jax.experimental import pallas as pl
from jax.experimental.pallas import tpu as pltpu
from jax.experimental.pallas import tpu_sc as plsc

N_NODES_C = 10000
N_EDGES_C = 320000
D_IN = 128
NC = 2
NS = 16
NHALF = N_NODES_C // NC
ACC_ROWS = 5120
EDGES_PER_TILE = N_EDGES_C // NS
BLK = 80
NBLK = EDGES_PER_TILE // BLK
NPAIR = NBLK // 2 + 1
MCAP = (2 * NPAIR + 2) * BLK


_GDN = lax.GatherDimensionNumbers(
    offset_dims=(), collapsed_slice_dims=(0,), start_index_map=(0,))


def _splat_lane(vec, lane):
    idx = jnp.full((16, 1), lane, dtype=jnp.int32)
    return lax.gather(vec, idx, _GDN, slice_sizes=(1,),
                      mode=lax.GatherScatterMode.PROMISE_IN_BOUNDS)


CB = 25
NCHUNK = EDGES_PER_TILE // (CB * BLK)
MBUF = (CB + 2) * BLK


def _sc_body(x_hbm, src_hbm, dst_hbm, u_hbm, zeros_hbm,
             accx_out, accux_out,
             srcb_v, dstb_v, ub_v,
             dstl0_v, dstl1_v, rows0_v, rows1_v, ux_v,
             gsem0, gsem1, ssem0, ssem1, usem,
             accx_sh, accux_sh):
    c = lax.axis_index("c")
    s = lax.axis_index("s")
    slice_rows = ACC_ROWS // NS
    r0 = s * slice_rows
    pltpu.sync_copy(zeros_hbm.at[pl.ds(r0, slice_rows)],
                    accx_sh.at[pl.ds(r0, slice_rows)])
    pltpu.sync_copy(zeros_hbm.at[pl.ds(r0, slice_rows)],
                    accux_sh.at[pl.ds(r0, slice_rows)])
    base_e = s * EDGES_PER_TILE
    half_lo = c * NHALF
    for p in range((MBUF - CB * BLK) // 16):
        o = CB * BLK + p * 16
        srcb_v[pl.ds(o, 16)] = jnp.zeros((16,), jnp.int32)
        dstb_v[pl.ds(o, 16)] = jnp.zeros((16,), jnp.int32) + half_lo + NHALF
        ub_v[pl.ds(o, 16)] = jnp.zeros((16,), jnp.float32)
    plsc.subcore_barrier()

    bufs = ((rows0_v, gsem0), (rows1_v, gsem1))

    def chunk(ch, carry):
        e0 = pl.multiple_of(base_e + ch * CB * BLK, BLK)
        pltpu.sync_copy(src_hbm.at[pl.ds(e0, CB * BLK)],
                        srcb_v.at[pl.ds(0, CB * BLK)])
        pltpu.sync_copy(dst_hbm.at[pl.ds(e0, CB * BLK)],
                        dstb_v.at[pl.ds(0, CB * BLK)])
        pltpu.sync_copy(u_hbm.at[pl.ds(e0, CB * BLK)],
                        ub_v.at[pl.ds(0, CB * BLK)])
        def pair(ip, carry2):
            for b in (0, 1):
                o = 1 - b
                rows_v, gsem = bufs[b]
                rows_o, gsem_o = bufs[o]
                g = 2 * ip + b
                off = g * BLK
                pltpu.async_copy(x_hbm.at[srcb_v.at[pl.ds(off, BLK)]],
                                 rows_v, gsem).wait()
                for gg in range(BLK // 16):
                    d = dstb_v[pl.ds(off + gg * 16, 16)]
                    dl = d - half_lo
                    oob = (dl < 0) | (dl >= NHALF)
                    dstl0_v[pl.ds(gg * 16, 16)] = jnp.where(oob, NHALF, dl)
                for gg in range(BLK // 16):
                    uv = ub_v[pl.ds(off + gg * 16, 16)]
                    uv = jnp.clip(uv, 0.0, 1.0)
                    for l in range(16):
                        j = gg * 16 + l
                        uj = _splat_lane(uv, l)
                        for k in range(D_IN // 16):
                            ux_v[j, pl.ds(k * 16, 16)] = (
                                rows_v[j, pl.ds(k * 16, 16)] * uj)
                d1 = pltpu.async_copy(rows_v, accx_sh.at[dstl0_v], ssem0,
                                      add=True)
                d2 = pltpu.async_copy(ux_v, accux_sh.at[dstl0_v], ssem1,
                                      add=True)
                d1.wait()
                d2.wait()
            return carry2

        lax.fori_loop(0, (CB + 1) // 2, pair, 0)
        return carry

    lax.fori_loop(0, NCHUNK, chunk, 0)
    plsc.subcore_barrier()
    pltpu.sync_copy(accx_sh.at[pl.ds(r0, slice_rows)],
                    accx_out.at[c, pl.ds(r0, slice_rows)])
    pltpu.sync_copy(accux_sh.at[pl.ds(r0, slice_rows)],
                    accux_out.at[c, pl.ds(r0, slice_rows)])


def _deg_body(dst_hbm, zeros_hbm, ones_hbm,
              deg_out,
              dstb_v, dstl0_v, dstl1_v, ones_v, sem0, sem1, deg_sh):
    c = lax.axis_index("c")
    s = lax.axis_index("s")
    slice_rows = ACC_ROWS // NS
    r0 = s * slice_rows
    pltpu.sync_copy(ones_hbm, ones_v)
    pltpu.sync_copy(zeros_hbm.at[pl.ds(r0, slice_rows)],
                    deg_sh.at[pl.ds(r0, slice_rows)])
    base_e = s * EDGES_PER_TILE
    pltpu.sync_copy(dst_hbm.at[pl.ds(base_e, EDGES_PER_TILE)],
                    dstb_v.at[pl.ds(0, EDGES_PER_TILE)])
    half_lo = c * NHALF
    for p in range((MCAP - EDGES_PER_TILE) // 16):
        o = EDGES_PER_TILE + p * 16
        dstb_v[pl.ds(o, 16)] = jnp.zeros((16,), jnp.int32) + half_lo + NHALF
    plsc.subcore_barrier()

    for gg in range(BLK // 16):
        dstl0_v[pl.ds(gg * 16, 16)] = jnp.full((16,), NHALF, jnp.int32)
        dstl1_v[pl.ds(gg * 16, 16)] = jnp.full((16,), NHALF, jnp.int32)
    pltpu.async_copy(ones_v, deg_sh.at[dstl0_v], sem0, add=True)
    pltpu.async_copy(ones_v, deg_sh.at[dstl1_v], sem1, add=True)

    bufs = ((dstl0_v, sem0), (dstl1_v, sem1))

    def pair(i2, carry):
        for b in (0, 1):
            dstl_v, sem = bufs[b]
            g = 2 * i2 + b
            off = pl.multiple_of(g * BLK, BLK)
            pltpu.make_async_copy(ones_v, deg_sh.at[dstl_v], sem).wait()
            for gg in range(BLK // 16):
                d = dstb_v[pl.ds(off + gg * 16, 16)]
                dl = d - half_lo
                oob = (dl < 0) | (dl >= NHALF)
                dstl_v[pl.ds(gg * 16, 16)] = jnp.where(oob, NHALF, dl)
            pltpu.async_copy(ones_v, deg_sh.at[dstl_v], sem, add=True)
        return carry

    lax.fori_loop(0, NPAIR, pair, 0)
    pltpu.make_async_copy(ones_v, deg_sh.at[dstl0_v], sem0).wait()
    pltpu.make_async_copy(ones_v, deg_sh.at[dstl1_v], sem1).wait()
    plsc.subcore_barrier()
    pltpu.sync_copy(deg_sh.at[pl.ds(r0, slice_rows)],
                    deg_out.at[c, pl.ds(r0, slice_rows)])


@jax.jit
def _sc_aggregate(x, src, dst, u, zeros, ones):
    mesh = plsc.VectorSubcoreMesh(core_axis_name="c", subcore_axis_name="s")
    acc_t = jax.ShapeDtypeStruct((NC, ACC_ROWS, D_IN), jnp.float32)
    main = pl.kernel(
        _sc_body,
        out_type=[acc_t, acc_t],
        mesh=mesh,
        scratch_types=[
            pltpu.VMEM((MBUF,), jnp.int32),
            pltpu.VMEM((MBUF,), jnp.int32),
            pltpu.VMEM((MBUF,), jnp.float32),
            pltpu.VMEM((BLK,), jnp.int32),
            pltpu.VMEM((BLK,), jnp.int32),
            pltpu.VMEM((BLK, D_IN), jnp.float32),
            pltpu.VMEM((BLK, D_IN), jnp.float32),
            pltpu.VMEM((BLK, D_IN), jnp.float32),
            pltpu.SemaphoreType.DMA,
            pltpu.SemaphoreType.DMA,
            pltpu.SemaphoreType.DMA,
            pltpu.SemaphoreType.DMA,
            pltpu.SemaphoreType.DMA,
            pltpu.VMEM_SHARED((ACC_ROWS, D_IN), jnp.float32),
            pltpu.VMEM_SHARED((ACC_ROWS, D_IN), jnp.float32),
        ],
    )
    accx, accux = main(x, src, dst, u, zeros)
    degk = pl.kernel(
        _deg_body,
        out_type=[acc_t],
        mesh=mesh,
        scratch_types=[
            pltpu.VMEM((MCAP,), jnp.int32),
            pltpu.VMEM((BLK,), jnp.int32),
            pltpu.VMEM((BLK,), jnp.int32),
            pltpu.VMEM((BLK, D_IN), jnp.float32),
            pltpu.SemaphoreType.DMA,
            pltpu.SemaphoreType.DMA,
            pltpu.VMEM_SHARED((ACC_ROWS, D_IN), jnp.float32),
        ],
    )
    (deg,) = degk(dst, zeros, ones)
    return accx, accux, deg


def _tc_body(sx_ref, sux_ref, deg_ref, x_ref, w0_ref, w1m0_ref, rw_ref,
             o_ref):
    sx = sx_ref[...]
    su = sux_ref[...]
    deg = deg_ref[:, 0:1]
    m = jnp.dot(sx, w0_ref[...], preferred_element_type=jnp.float32,
                precision=lax.Precision.HIGHEST)
    m += jnp.dot(su, w1m0_ref[...], preferred_element_type=jnp.float32,
                 precision=lax.Precision.HIGHEST)
    root = jnp.dot(x_ref[...], rw_ref[...], preferred_element_type=jnp.float32,
                   precision=lax.Precision.HIGHEST)
    o_ref[...] = m / jnp.maximum(deg, 1.0) + root


@jax.jit
def _tc_combine(sx, sux, deg, x, w0, w1m0, root_w):
    rows = 1000
    grid = N_NODES_C // rows
    return pl.pallas_call(
        _tc_body,
        grid=(grid,),
        in_specs=[
            pl.BlockSpec((rows, D_IN), lambda i: (i, 0)),
            pl.BlockSpec((rows, D_IN), lambda i: (i, 0)),
            pl.BlockSpec((rows, D_IN), lambda i: (i, 0)),
            pl.BlockSpec((rows, D_IN), lambda i: (i, 0)),
            pl.BlockSpec((D_IN, D_IN), lambda i: (0, 0)),
            pl.BlockSpec((D_IN, D_IN), lambda i: (0, 0)),
            pl.BlockSpec((D_IN, D_IN), lambda i: (0, 0)),
        ],
        out_specs=pl.BlockSpec((rows, D_IN), lambda i: (i, 0)),
        out_shape=jax.ShapeDtypeStruct((N_NODES_C, D_IN), jnp.float32),
    )(sx, sux, deg, x, w0, w1m0, root_w)


def kernel(node_feature, edge_index, edge_feature, W, root_W):
    src = edge_index[0].astype(jnp.int32)
    dst = edge_index[1].astype(jnp.int32)
    u = edge_feature[:, 0]
    zeros = jnp.zeros((ACC_ROWS, D_IN), jnp.float32)
    ones = jnp.ones((BLK, D_IN), jnp.float32)
    accx, accux, accdeg = _sc_aggregate(node_feature, src, dst, u, zeros,
                                        ones)
    sx = accx[:, :NHALF, :].reshape(N_NODES_C, D_IN)
    sux = accux[:, :NHALF, :].reshape(N_NODES_C, D_IN)
    deg = accdeg[:, :NHALF, :].reshape(N_NODES_C, D_IN)
    return _tc_combine(sx, sux, deg, node_feature,
                       W[0], W[1] - W[0], root_W)


# speedup vs baseline: 2.8585x; 1.8250x over previous
import jax
import jax.numpy as jnp
from jax import lax
from jax.experimental import pallas as pl
from jax.experimental.pallas import tpu as pltpu
from jax.experimental.pallas import tpu_sc as plsc

N_NODES_C = 10000
N_EDGES_C = 320000
D_IN = 128
NC = 2
NS = 16
NHALF = N_NODES_C // NC
ACC_ROWS = 5120
EDGES_PER_TILE = N_EDGES_C // NS
BLK = 128
NFULL = EDGES_PER_TILE // BLK
TAIL = EDGES_PER_TILE - NFULL * BLK
TOFF = NFULL * BLK


_GDN = lax.GatherDimensionNumbers(
    offset_dims=(), collapsed_slice_dims=(0,), start_index_map=(0,))


def _splat_lane(vec, lane):
    idx = jnp.full((16, 1), lane, dtype=jnp.int32)
    return lax.gather(vec, idx, _GDN, slice_sizes=(1,),
                      mode=lax.GatherScatterMode.PROMISE_IN_BOUNDS)


CB = 6
NCHUNK = NFULL // CB
MBUF = CB * BLK


def _sc_body(x_hbm, src_hbm, dst_hbm, u_hbm, zeros_hbm,
             accx_out, accux_out,
             srcb_v, dstb_v, ub_v,
             dstl_v, dstlt_v, rows_v, ux_v,
             gsem0, ssem0, ssem1,
             accx_sh, accux_sh):
    c = lax.axis_index("c")
    s = lax.axis_index("s")
    slice_rows = ACC_ROWS // NS
    r0 = s * slice_rows
    pltpu.sync_copy(zeros_hbm.at[pl.ds(r0, slice_rows)],
                    accx_sh.at[pl.ds(r0, slice_rows)])
    pltpu.sync_copy(zeros_hbm.at[pl.ds(r0, slice_rows)],
                    accux_sh.at[pl.ds(r0, slice_rows)])
    base_e = s * EDGES_PER_TILE
    half_lo = c * NHALF
    plsc.subcore_barrier()

    def chunk(ch, carry):
        e0 = pl.multiple_of(base_e + ch * CB * BLK, BLK)
        pltpu.sync_copy(src_hbm.at[pl.ds(e0, CB * BLK)],
                        srcb_v.at[pl.ds(0, CB * BLK)])
        pltpu.sync_copy(dst_hbm.at[pl.ds(e0, CB * BLK)],
                        dstb_v.at[pl.ds(0, CB * BLK)])
        pltpu.sync_copy(u_hbm.at[pl.ds(e0, CB * BLK)],
                        ub_v.at[pl.ds(0, CB * BLK)])
        def block(g, carry2):
            off = g * BLK
            for gg in range(BLK // 16):
                d = dstb_v[pl.ds(off + gg * 16, 16)]
                dl = d - half_lo
                oob = (dl < 0) | (dl >= NHALF)
                dstl_v[pl.ds(gg * 16, 16)] = jnp.where(oob, NHALF, dl)
            pltpu.async_copy(
                x_hbm.at[srcb_v.at[pl.ds(off, BLK)]], rows_v,
                gsem0).wait()
            for gg in range(BLK // 16):
                uv = ub_v[pl.ds(off + gg * 16, 16)]
                uv = jnp.clip(uv, 0.0, 1.0)
                for l in range(16):
                    j = gg * 16 + l
                    uj = _splat_lane(uv, l)
                    for k in range(D_IN // 16):
                        ux_v[j, pl.ds(k * 16, 16)] = (
                            rows_v[j, pl.ds(k * 16, 16)] * uj)
            pltpu.sync_copy(rows_v, accx_sh.at[dstl_v], add=True)
            pltpu.sync_copy(ux_v, accux_sh.at[dstl_v], add=True)
            return carry2

        lax.fori_loop(0, CB, block, 0)
        return carry

    lax.fori_loop(0, NCHUNK, chunk, 0)
    t0 = pl.multiple_of(base_e + TOFF, 8)
    pltpu.sync_copy(src_hbm.at[pl.ds(t0, TAIL)],
                    srcb_v.at[pl.ds(0, TAIL)])
    pltpu.sync_copy(dst_hbm.at[pl.ds(t0, TAIL)],
                    dstb_v.at[pl.ds(0, TAIL)])
    pltpu.sync_copy(u_hbm.at[pl.ds(t0, TAIL)], ub_v.at[pl.ds(0, TAIL)])
    for gg in range(TAIL // 16):
        d = dstb_v[pl.ds(gg * 16, 16)]
        dl = d - half_lo
        oob = (dl < 0) | (dl >= NHALF)
        dstlt_v[pl.ds(gg * 16, 16)] = jnp.where(oob, NHALF, dl)
    pltpu.async_copy(
        x_hbm.at[srcb_v.at[pl.ds(0, TAIL)]],
        rows_v.at[pl.ds(0, TAIL)], gsem0).wait()
    for gg in range(TAIL // 16):
        uv = ub_v[pl.ds(gg * 16, 16)]
        uv = jnp.clip(uv, 0.0, 1.0)
        for l in range(16):
            j = gg * 16 + l
            uj = _splat_lane(uv, l)
            for k in range(D_IN // 16):
                ux_v[j, pl.ds(k * 16, 16)] = (
                    rows_v[j, pl.ds(k * 16, 16)] * uj)
    pltpu.sync_copy(rows_v.at[pl.ds(0, TAIL)],
                    accx_sh.at[dstlt_v], add=True)
    pltpu.sync_copy(ux_v.at[pl.ds(0, TAIL)],
                    accux_sh.at[dstlt_v], add=True)
    plsc.subcore_barrier()
    pltpu.sync_copy(accx_sh.at[pl.ds(r0, slice_rows)],
                    accx_out.at[c, pl.ds(r0, slice_rows)])
    pltpu.sync_copy(accux_sh.at[pl.ds(r0, slice_rows)],
                    accux_out.at[c, pl.ds(r0, slice_rows)])


def _deg_body(dst_hbm, zeros_hbm, ones_hbm,
              deg_out,
              dstb_v, dstl_v, dstlt_v, ones_v, sem0, deg_sh):
    c = lax.axis_index("c")
    s = lax.axis_index("s")
    slice_rows = ACC_ROWS // NS
    r0 = s * slice_rows
    pltpu.sync_copy(ones_hbm, ones_v)
    pltpu.sync_copy(zeros_hbm.at[pl.ds(r0, slice_rows)],
                    deg_sh.at[pl.ds(r0, slice_rows)])
    base_e = s * EDGES_PER_TILE
    pltpu.sync_copy(dst_hbm.at[pl.ds(base_e, EDGES_PER_TILE)], dstb_v)
    half_lo = c * NHALF
    plsc.subcore_barrier()

    def block(g, carry):
        off = g * BLK
        for gg in range(BLK // 16):
            d = dstb_v[pl.ds(off + gg * 16, 16)]
            dl = d - half_lo
            oob = (dl < 0) | (dl >= NHALF)
            dstl_v[pl.ds(gg * 16, 16)] = jnp.where(oob, NHALF, dl)
        pltpu.sync_copy(ones_v, deg_sh.at[dstl_v], add=True)
        return carry

    lax.fori_loop(0, NFULL, block, 0)
    for gg in range(TAIL // 16):
        d = dstb_v[pl.ds(TOFF + gg * 16, 16)]
        dl = d - half_lo
        oob = (dl < 0) | (dl >= NHALF)
        dstlt_v[pl.ds(gg * 16, 16)] = jnp.where(oob, NHALF, dl)
    pltpu.sync_copy(ones_v.at[pl.ds(0, TAIL)],
                    deg_sh.at[dstlt_v], add=True)
    plsc.subcore_barrier()
    pltpu.sync_copy(deg_sh.at[pl.ds(r0, slice_rows)],
                    deg_out.at[c, pl.ds(r0, slice_rows)])


@jax.jit
def _sc_aggregate(x, src, dst, u, zeros, ones):
    mesh = plsc.VectorSubcoreMesh(core_axis_name="c", subcore_axis_name="s")
    acc_t = jax.ShapeDtypeStruct((NC, ACC_ROWS, D_IN), jnp.float32)
    main = pl.kernel(
        _sc_body,
        out_type=[acc_t, acc_t],
        mesh=mesh,
        scratch_types=[
            pltpu.VMEM((MBUF,), jnp.int32),
            pltpu.VMEM((MBUF,), jnp.int32),
            pltpu.VMEM((MBUF,), jnp.float32),
            pltpu.VMEM((BLK,), jnp.int32),
            pltpu.VMEM((TAIL,), jnp.int32),
            pltpu.VMEM((BLK, D_IN), jnp.float32),
            pltpu.VMEM((BLK, D_IN), jnp.float32),
            pltpu.SemaphoreType.DMA,
            pltpu.SemaphoreType.DMA,
            pltpu.SemaphoreType.DMA,
            pltpu.VMEM_SHARED((ACC_ROWS, D_IN), jnp.float32),
            pltpu.VMEM_SHARED((ACC_ROWS, D_IN), jnp.float32),
        ],
    )
    accx, accux = main(x, src, dst, u, zeros)
    degk = pl.kernel(
        _deg_body,
        out_type=[acc_t],
        mesh=mesh,
        scratch_types=[
            pltpu.VMEM((EDGES_PER_TILE,), jnp.int32),
            pltpu.VMEM((BLK,), jnp.int32),
            pltpu.VMEM((TAIL,), jnp.int32),
            pltpu.VMEM((BLK, D_IN), jnp.float32),
            pltpu.SemaphoreType.DMA,
            pltpu.VMEM_SHARED((ACC_ROWS, D_IN), jnp.float32),
        ],
    )
    (deg,) = degk(dst, zeros, ones)
    return accx, accux, deg


def _tc_body(sx_ref, sux_ref, deg_ref, x_ref, w0_ref, w1m0_ref, rw_ref,
             o_ref):
    sx = sx_ref[...]
    su = sux_ref[...]
    deg = deg_ref[:, 0:1]
    m = jnp.dot(sx, w0_ref[...], preferred_element_type=jnp.float32,
                precision=lax.Precision.HIGHEST)
    m += jnp.dot(su, w1m0_ref[...], preferred_element_type=jnp.float32,
                 precision=lax.Precision.HIGHEST)
    root = jnp.dot(x_ref[...], rw_ref[...], preferred_element_type=jnp.float32,
                   precision=lax.Precision.HIGHEST)
    o_ref[...] = m / jnp.maximum(deg, 1.0) + root


@jax.jit
def _tc_combine(sx, sux, deg, x, w0, w1m0, root_w):
    rows = 1000
    grid = N_NODES_C // rows
    return pl.pallas_call(
        _tc_body,
        grid=(grid,),
        in_specs=[
            pl.BlockSpec((rows, D_IN), lambda i: (i, 0)),
            pl.BlockSpec((rows, D_IN), lambda i: (i, 0)),
            pl.BlockSpec((rows, D_IN), lambda i: (i, 0)),
            pl.BlockSpec((rows, D_IN), lambda i: (i, 0)),
            pl.BlockSpec((D_IN, D_IN), lambda i: (0, 0)),
            pl.BlockSpec((D_IN, D_IN), lambda i: (0, 0)),
            pl.BlockSpec((D_IN, D_IN), lambda i: (0, 0)),
        ],
        out_specs=pl.BlockSpec((rows, D_IN), lambda i: (i, 0)),
        out_shape=jax.ShapeDtypeStruct((N_NODES_C, D_IN), jnp.float32),
    )(sx, sux, deg, x, w0, w1m0, root_w)


def kernel(node_feature, edge_index, edge_feature, W, root_W):
    src = edge_index[0].astype(jnp.int32)
    dst = edge_index[1].astype(jnp.int32)
    u = edge_feature[:, 0]
    zeros = jnp.zeros((ACC_ROWS, D_IN), jnp.float32)
    ones = jnp.ones((BLK, D_IN), jnp.float32)
    accx, accux, accdeg = _sc_aggregate(node_feature, src, dst, u, zeros,
                                        ones)
    sx = accx[:, :NHALF, :].reshape(N_NODES_C, D_IN)
    sux = accux[:, :NHALF, :].reshape(N_NODES_C, D_IN)
    deg = accdeg[:, :NHALF, :].reshape(N_NODES_C, D_IN)
    return _tc_combine(sx, sux, deg, node_feature,
                       W[0], W[1] - W[0], root_W)
